# baseline (speedup 1.0000x reference)
import functools

import jax
from jax import lax
import jax.numpy as jnp
from jax.experimental import pallas as pl
from jax.experimental.pallas import tpu as pltpu
from jax.experimental.pallas import tpu_sc as plsc

_HI = jax.lax.Precision.HIGHEST

_NC = 2
_NS = 16
_NW = _NC * _NS
_LANES = 16
_SC_ROWS = 1024
_RPW = _SC_ROWS // _NW
_GRP = 4


def _lane_gather(x, idx):
    return lax.gather(
        x, idx[:, None],
        lax.GatherDimensionNumbers(offset_dims=(), collapsed_slice_dims=(0,),
                                   start_index_map=(0,)),
        (1,), mode=lax.GatherScatterMode.PROMISE_IN_BOUNDS)


def _sc_qk_kernel(wq_hbm, bq_hbm, wk_hbm, out_hbm,
                  wq_v, wk_v, bq_v, acc_v, sem, *, D):
    wid = lax.axis_index("s") * _NC + lax.axis_index("c")
    base = wid * _RPW
    nch = D // _LANES

    pltpu.sync_copy(bq_hbm.at[pl.ds(base, _RPW)], bq_v)

    def zero_body(j, _):
        acc_v[pl.ds(j * _LANES, _LANES)] = jnp.zeros((_LANES,), jnp.float32)
        return 0
    lax.fori_loop(0, nch, zero_body, 0)

    lanes_iota = lax.iota(jnp.int32, _LANES)

    def allsum(x):
        for k in (1, 2, 4, 8):
            x = x + _lane_gather(x, lanes_iota ^ k)
        return x

    def group_body(g, _):
        row0 = base + g * _GRP
        copies = []
        for b in range(_GRP):
            copies.append(pltpu.async_copy(wq_hbm.at[row0 + b], wq_v.at[b], sem))
            copies.append(pltpu.async_copy(wk_hbm.at[row0 + b], wk_v.at[b], sem))
        for cp in copies:
            cp.wait()
        for b in range(_GRP):
            def sum_body(j, a):
                return a + wq_v[b, pl.ds(j * _LANES, _LANES)]
            a16 = lax.fori_loop(0, nch, sum_body,
                                jnp.zeros((_LANES,), jnp.float32))
            li = g * _GRP + b
            bq16 = bq_v[pl.ds((li // _LANES) * _LANES, _LANES)]
            bqv = _lane_gather(bq16, jnp.full((_LANES,), li % _LANES,
                                              jnp.int32))
            qc = allsum(a16) + bqv

            def upd_body(j, _):
                sl = pl.ds(j * _LANES, _LANES)
                plsc.addupdate(acc_v.at[sl], qc * wk_v[b, sl])
                return 0
            lax.fori_loop(0, nch, upd_body, 0)
        return 0

    lax.fori_loop(0, _RPW // _GRP, group_body, 0)
    pltpu.sync_copy(acc_v, out_hbm.at[wid])


def _sc_qk_partials(Wq, bq, Wk):
    D = Wq.shape[1]
    mesh = plsc.VectorSubcoreMesh(core_axis_name="c", subcore_axis_name="s")
    return functools.partial(
        pl.kernel,
        mesh=mesh,
        out_type=jax.ShapeDtypeStruct((_NW, D), jnp.float32),
        scratch_types=[
            pltpu.VMEM((_GRP, D), jnp.float32),
            pltpu.VMEM((_GRP, D), jnp.float32),
            pltpu.VMEM((_RPW,), jnp.float32),
            pltpu.VMEM((D,), jnp.float32),
            pltpu.SemaphoreType.DMA,
        ],
    )(functools.partial(_sc_qk_kernel, D=D))(Wq, bq, Wk)


def _qk_kernel(wq_ref, bq_ref, wk_ref, qk_ref):
    i = pl.program_id(0)
    qc = jnp.sum(wq_ref[...], axis=1) + bq_ref[0, :]
    part = jnp.sum(qc[:, None] * wk_ref[...], axis=0, keepdims=True)

    @pl.when(i == 0)
    def _():
        qk_ref[...] = jnp.zeros_like(qk_ref)

    qk_ref[...] += part


def _flash_kernel(h_ref, qk_ref, qksc_ref, hbar_ref,
                  acc_ref, m_ref, s_ref, qkf_ref, *, inv_scale):
    i = pl.program_id(0)
    nsteps = pl.num_programs(0)

    @pl.when(i == 0)
    def _():
        m_ref[...] = jnp.full_like(m_ref, -jnp.inf)
        s_ref[...] = jnp.zeros_like(s_ref)
        acc_ref[...] = jnp.zeros_like(acc_ref)
        qkf_ref[...] = qk_ref[...] + jnp.sum(qksc_ref[...], axis=0,
                                             keepdims=True)

    h = h_ref[...]
    qk = qkf_ref[0, :]
    t = jax.lax.dot_general(
        h, qk, (((2,), (0,)), ((), ())),
        preferred_element_type=jnp.float32, precision=_HI)
    t = t * inv_scale

    m_prev = m_ref[...]
    m_new = jnp.maximum(m_prev, jnp.max(t, axis=1, keepdims=True))
    alpha = jnp.exp(m_prev - m_new)
    p = jnp.exp(t - m_new)
    s_ref[...] = s_ref[...] * alpha + jnp.sum(p, axis=1, keepdims=True)
    pv = jax.lax.dot_general(
        p, h, (((1,), (1,)), ((0,), (0,))),
        preferred_element_type=jnp.float32)
    acc_ref[...] = acc_ref[...] * alpha + pv
    m_ref[...] = m_new

    @pl.when(i == nsteps - 1)
    def _():
        hbar_ref[...] = acc_ref[...] / s_ref[...]


def _tail_kernel(hbar_ref, wv_ref, bv_ref, we_ref, be_ref,
                 expert_ref, pmax_ref, logits_ref):
    i = pl.program_id(0)
    nsteps = pl.num_programs(0)
    hb = hbar_ref[...]
    hb_hi = hb.astype(jnp.bfloat16)
    hb_lo = (hb - hb_hi.astype(jnp.float32)).astype(jnp.bfloat16)
    hb2 = jnp.concatenate([hb_hi, hb_lo], axis=0)
    wv_hi = wv_ref[...].astype(jnp.bfloat16)
    bdim = hb.shape[0]
    rr = jax.lax.dot_general(
        hb2, wv_hi, (((1,), (1,)), ((), ())),
        preferred_element_type=jnp.float32)
    r = rr[:bdim, :] + rr[bdim:, :] + bv_ref[...]
    part = jax.lax.dot_general(
        r, we_ref[...], (((1,), (1,)), ((), ())),
        preferred_element_type=jnp.float32, precision=_HI)

    @pl.when(i == 0)
    def _():
        logits_ref[...] = jnp.zeros_like(logits_ref)

    logits_ref[...] += part

    @pl.when(i == nsteps - 1)
    def _():
        logits = logits_ref[...] + be_ref[...]
        logits_ref[...] = logits
        row_max = jnp.max(logits, axis=1, keepdims=True)
        ex = jnp.exp(logits - row_max)
        denom = jnp.sum(ex, axis=1, keepdims=True)
        pmax_ref[...] = jnp.max(ex, axis=1, keepdims=True) / denom
        bdim, edim = logits.shape
        idx = jax.lax.broadcasted_iota(jnp.int32, (bdim, edim), 1)
        am = jnp.min(jnp.where(logits == row_max, idx, edim),
                     axis=1, keepdims=True)
        expert_ref[...] = (idx == am).astype(jnp.int32)


def kernel(h_dense, Wq, bq, Wk, bk, Wv, bv, We, be):
    del bk
    B, N, D = h_dense.shape
    E = We.shape[0]
    f32 = jnp.float32

    qk_sc = _sc_qk_partials(Wq, bq, Wk)

    C1 = 512
    off1 = _SC_ROWS // C1
    qk = pl.pallas_call(
        _qk_kernel,
        grid=((D - _SC_ROWS) // C1,),
        in_specs=[
            pl.BlockSpec((C1, D), lambda i: (i + off1, 0)),
            pl.BlockSpec((1, C1), lambda i: (0, i + off1)),
            pl.BlockSpec((C1, D), lambda i: (i + off1, 0)),
        ],
        out_specs=pl.BlockSpec((1, D), lambda i: (0, 0)),
        out_shape=jax.ShapeDtypeStruct((1, D), f32),
    )(Wq, bq.reshape(1, D), Wk)

    C2 = 256
    hbar = pl.pallas_call(
        functools.partial(_flash_kernel, inv_scale=1.0 / (float(D) ** 0.5)),
        grid=(N // C2,),
        in_specs=[
            pl.BlockSpec((B, C2, D), lambda i: (0, i, 0)),
            pl.BlockSpec((1, D), lambda i: (0, 0)),
            pl.BlockSpec((_NW, D), lambda i: (0, 0)),
        ],
        out_specs=pl.BlockSpec((B, D), lambda i: (0, 0)),
        out_shape=jax.ShapeDtypeStruct((B, D), f32),
        scratch_shapes=[
            pltpu.VMEM((B, D), f32),
            pltpu.VMEM((B, 1), f32),
            pltpu.VMEM((B, 1), f32),
            pltpu.VMEM((1, D), f32),
        ],
    )(h_dense, qk, qk_sc)

    C3 = 512
    expert, pmax, logits = pl.pallas_call(
        _tail_kernel,
        grid=(D // C3,),
        in_specs=[
            pl.BlockSpec((B, D), lambda i: (0, 0)),
            pl.BlockSpec((C3, D), lambda i: (i, 0)),
            pl.BlockSpec((1, C3), lambda i: (0, i)),
            pl.BlockSpec((E, C3), lambda i: (0, i)),
            pl.BlockSpec((1, E), lambda i: (0, 0)),
        ],
        out_specs=[
            pl.BlockSpec((B, E), lambda i: (0, 0)),
            pl.BlockSpec((B, 1), lambda i: (0, 0)),
            pl.BlockSpec((B, E), lambda i: (0, 0)),
        ],
        out_shape=[
            jax.ShapeDtypeStruct((B, E), jnp.int32),
            jax.ShapeDtypeStruct((B, 1), f32),
            jax.ShapeDtypeStruct((B, E), f32),
        ],
    )(hbar, Wv, bv.reshape(1, D), We, be.reshape(1, E))

    return (expert, pmax, logits)


# speedup vs baseline: 1.3973x; 1.3973x over previous
import functools

import jax
from jax import lax
import jax.numpy as jnp
from jax.experimental import pallas as pl
from jax.experimental.pallas import tpu as pltpu
from jax.experimental.pallas import tpu_sc as plsc

_HI = jax.lax.Precision.HIGHEST

_NC = 2
_NS = 16
_NW = _NC * _NS
_LANES = 16
_SC_ROWS = 512
_RPW = _SC_ROWS // _NW
_GRP = 4


def _lane_gather(x, idx):
    return lax.gather(
        x, idx[:, None],
        lax.GatherDimensionNumbers(offset_dims=(), collapsed_slice_dims=(0,),
                                   start_index_map=(0,)),
        (1,), mode=lax.GatherScatterMode.PROMISE_IN_BOUNDS)


def _sc_qk_kernel(wq_hbm, bq_hbm, wk_hbm, out_hbm,
                  wq_v, wk_v, bq_v, acc_v, sem, *, D):
    wid = lax.axis_index("s") * _NC + lax.axis_index("c")
    base = wid * _RPW
    nch = D // _LANES

    pltpu.sync_copy(bq_hbm.at[pl.ds(base, _RPW)], bq_v)

    def zero_body(j, _):
        for u in range(4):
            acc_v[pl.ds((j * 4 + u) * _LANES, _LANES)] = jnp.zeros(
                (_LANES,), jnp.float32)
        return 0
    lax.fori_loop(0, nch // 4, zero_body, 0)

    lanes_iota = lax.iota(jnp.int32, _LANES)

    def allsum(x):
        for k in (1, 2, 4, 8):
            x = x + _lane_gather(x, lanes_iota ^ k)
        return x

    def group_body(g, _):
        row0 = base + g * _GRP
        copies = []
        for b in range(_GRP):
            copies.append(pltpu.async_copy(wq_hbm.at[row0 + b], wq_v.at[b], sem))
            copies.append(pltpu.async_copy(wk_hbm.at[row0 + b], wk_v.at[b], sem))
        for cp in copies:
            cp.wait()
        for b in range(_GRP):
            def sum_body(j, a):
                for u in range(4):
                    a = a + wq_v[b, pl.ds((j * 4 + u) * _LANES, _LANES)]
                return a
            a16 = lax.fori_loop(0, nch // 4, sum_body,
                                jnp.zeros((_LANES,), jnp.float32))
            li = g * _GRP + b
            bq16 = bq_v[pl.ds((li // _LANES) * _LANES, _LANES)]
            bqv = _lane_gather(bq16, jnp.full((_LANES,), li % _LANES,
                                              jnp.int32))
            qc = allsum(a16) + bqv

            def upd_body(j, _):
                for u in range(4):
                    sl = pl.ds((j * 4 + u) * _LANES, _LANES)
                    plsc.addupdate(acc_v.at[sl], qc * wk_v[b, sl])
                return 0
            lax.fori_loop(0, nch // 4, upd_body, 0)
        return 0

    lax.fori_loop(0, _RPW // _GRP, group_body, 0)
    pltpu.sync_copy(acc_v, out_hbm.at[wid])


def _sc_qk_partials(Wq, bq, Wk):
    D = Wq.shape[1]
    mesh = plsc.VectorSubcoreMesh(core_axis_name="c", subcore_axis_name="s",
                                  num_cores=_NC)
    return functools.partial(
        pl.kernel,
        mesh=mesh,
        out_type=jax.ShapeDtypeStruct((_NW, D), jnp.float32),
        scratch_types=[
            pltpu.VMEM((_GRP, D), jnp.float32),
            pltpu.VMEM((_GRP, D), jnp.float32),
            pltpu.VMEM((_RPW,), jnp.float32),
            pltpu.VMEM((D,), jnp.float32),
            pltpu.SemaphoreType.DMA,
        ],
    )(functools.partial(_sc_qk_kernel, D=D))(Wq, bq, Wk)


def _qk_kernel(wq_ref, bq_ref, wk_ref, qk_ref):
    i = pl.program_id(0)
    qc = jnp.sum(wq_ref[...], axis=1) + bq_ref[0, :]
    part = jnp.sum(qc[:, None] * wk_ref[...], axis=0, keepdims=True)

    @pl.when(i == 0)
    def _():
        qk_ref[...] = jnp.zeros_like(qk_ref)

    qk_ref[...] += part


def _flash_kernel(h_ref, qk_ref, qksc_ref, hbar_ref,
                  acc_ref, m_ref, s_ref, qkf_ref, *, inv_scale):
    i = pl.program_id(0)
    nsteps = pl.num_programs(0)

    @pl.when(i == 0)
    def _():
        m_ref[...] = jnp.full_like(m_ref, -jnp.inf)
        s_ref[...] = jnp.zeros_like(s_ref)
        acc_ref[...] = jnp.zeros_like(acc_ref)
        qkf_ref[...] = qk_ref[...] + jnp.sum(qksc_ref[...], axis=0,
                                             keepdims=True)

    h = h_ref[...]
    qk = qkf_ref[0, :]
    t = jax.lax.dot_general(
        h, qk, (((2,), (0,)), ((), ())),
        preferred_element_type=jnp.float32, precision=_HI)
    t = t * inv_scale

    m_prev = m_ref[...]
    m_new = jnp.maximum(m_prev, jnp.max(t, axis=1, keepdims=True))
    alpha = jnp.exp(m_prev - m_new)
    p = jnp.exp(t - m_new)
    s_ref[...] = s_ref[...] * alpha + jnp.sum(p, axis=1, keepdims=True)
    pv = jax.lax.dot_general(
        p, h, (((1,), (1,)), ((0,), (0,))),
        preferred_element_type=jnp.float32)
    acc_ref[...] = acc_ref[...] * alpha + pv
    m_ref[...] = m_new

    @pl.when(i == nsteps - 1)
    def _():
        hbar_ref[...] = acc_ref[...] / s_ref[...]


def _tail_kernel(hbar_ref, wv_ref, bv_ref, we_ref, be_ref,
                 expert_ref, pmax_ref, logits_ref):
    i = pl.program_id(0)
    nsteps = pl.num_programs(0)
    hb = hbar_ref[...]
    hb_hi = hb.astype(jnp.bfloat16)
    hb_lo = (hb - hb_hi.astype(jnp.float32)).astype(jnp.bfloat16)
    hb2 = jnp.concatenate([hb_hi, hb_lo], axis=0)
    wv_hi = wv_ref[...].astype(jnp.bfloat16)
    bdim = hb.shape[0]
    rr = jax.lax.dot_general(
        hb2, wv_hi, (((1,), (1,)), ((), ())),
        preferred_element_type=jnp.float32)
    r = rr[:bdim, :] + rr[bdim:, :] + bv_ref[...]
    part = jax.lax.dot_general(
        r, we_ref[...], (((1,), (1,)), ((), ())),
        preferred_element_type=jnp.float32, precision=_HI)

    @pl.when(i == 0)
    def _():
        logits_ref[...] = jnp.zeros_like(logits_ref)

    logits_ref[...] += part

    @pl.when(i == nsteps - 1)
    def _():
        logits = logits_ref[...] + be_ref[...]
        logits_ref[...] = logits
        row_max = jnp.max(logits, axis=1, keepdims=True)
        ex = jnp.exp(logits - row_max)
        denom = jnp.sum(ex, axis=1, keepdims=True)
        pmax_ref[...] = jnp.max(ex, axis=1, keepdims=True) / denom
        bdim, edim = logits.shape
        idx = jax.lax.broadcasted_iota(jnp.int32, (bdim, edim), 1)
        am = jnp.min(jnp.where(logits == row_max, idx, edim),
                     axis=1, keepdims=True)
        expert_ref[...] = (idx == am).astype(jnp.int32)


def kernel(h_dense, Wq, bq, Wk, bk, Wv, bv, We, be):
    del bk
    B, N, D = h_dense.shape
    E = We.shape[0]
    f32 = jnp.float32

    qk_sc = _sc_qk_partials(Wq, bq, Wk)

    C1 = 512
    off1 = _SC_ROWS // C1
    qk = pl.pallas_call(
        _qk_kernel,
        grid=((D - _SC_ROWS) // C1,),
        in_specs=[
            pl.BlockSpec((C1, D), lambda i: (i + off1, 0)),
            pl.BlockSpec((1, C1), lambda i: (0, i + off1)),
            pl.BlockSpec((C1, D), lambda i: (i + off1, 0)),
        ],
        out_specs=pl.BlockSpec((1, D), lambda i: (0, 0)),
        out_shape=jax.ShapeDtypeStruct((1, D), f32),
    )(Wq, bq.reshape(1, D), Wk)

    C2 = 256
    hbar = pl.pallas_call(
        functools.partial(_flash_kernel, inv_scale=1.0 / (float(D) ** 0.5)),
        grid=(N // C2,),
        in_specs=[
            pl.BlockSpec((B, C2, D), lambda i: (0, i, 0)),
            pl.BlockSpec((1, D), lambda i: (0, 0)),
            pl.BlockSpec((_NW, D), lambda i: (0, 0)),
        ],
        out_specs=pl.BlockSpec((B, D), lambda i: (0, 0)),
        out_shape=jax.ShapeDtypeStruct((B, D), f32),
        scratch_shapes=[
            pltpu.VMEM((B, D), f32),
            pltpu.VMEM((B, 1), f32),
            pltpu.VMEM((B, 1), f32),
            pltpu.VMEM((1, D), f32),
        ],
    )(h_dense, qk, qk_sc)

    C3 = 512
    expert, pmax, logits = pl.pallas_call(
        _tail_kernel,
        grid=(D // C3,),
        in_specs=[
            pl.BlockSpec((B, D), lambda i: (0, 0)),
            pl.BlockSpec((C3, D), lambda i: (i, 0)),
            pl.BlockSpec((1, C3), lambda i: (0, i)),
            pl.BlockSpec((E, C3), lambda i: (0, i)),
            pl.BlockSpec((1, E), lambda i: (0, 0)),
        ],
        out_specs=[
            pl.BlockSpec((B, E), lambda i: (0, 0)),
            pl.BlockSpec((B, 1), lambda i: (0, 0)),
            pl.BlockSpec((B, E), lambda i: (0, 0)),
        ],
        out_shape=[
            jax.ShapeDtypeStruct((B, E), jnp.int32),
            jax.ShapeDtypeStruct((B, 1), f32),
            jax.ShapeDtypeStruct((B, E), f32),
        ],
    )(hbar, Wv, bv.reshape(1, D), We, be.reshape(1, E))

    return (expert, pmax, logits)


# speedup vs baseline: 1.4551x; 1.0413x over previous
import functools

import jax
from jax import lax
import jax.numpy as jnp
from jax.experimental import pallas as pl
from jax.experimental.pallas import tpu as pltpu
from jax.experimental.pallas import tpu_sc as plsc

_HI = jax.lax.Precision.HIGHEST

_NC = 2
_NS = 16
_NW = _NC * _NS
_LANES = 16
_SC_ROWS = 256
_RPW = _SC_ROWS // _NW
_GRP = 4


def _lane_gather(x, idx):
    return lax.gather(
        x, idx[:, None],
        lax.GatherDimensionNumbers(offset_dims=(), collapsed_slice_dims=(0,),
                                   start_index_map=(0,)),
        (1,), mode=lax.GatherScatterMode.PROMISE_IN_BOUNDS)


def _sc_qk_kernel(wq_hbm, bq_hbm, wk_hbm, out_hbm,
                  wq_v, wk_v, bq_v, acc_v, sem, *, D):
    wid = lax.axis_index("s") * _NC + lax.axis_index("c")
    base = wid * _RPW
    nch = D // _LANES

    pltpu.sync_copy(bq_hbm.at[pl.ds(base, _RPW)], bq_v)

    def zero_body(j, _):
        for u in range(4):
            acc_v[pl.ds((j * 4 + u) * _LANES, _LANES)] = jnp.zeros(
                (_LANES,), jnp.float32)
        return 0
    lax.fori_loop(0, nch // 4, zero_body, 0)

    lanes_iota = lax.iota(jnp.int32, _LANES)

    def allsum(x):
        for k in (1, 2, 4, 8):
            x = x + _lane_gather(x, lanes_iota ^ k)
        return x

    def group_body(g, _):
        row0 = base + g * _GRP
        copies = []
        for b in range(_GRP):
            copies.append(pltpu.async_copy(wq_hbm.at[row0 + b], wq_v.at[b], sem))
            copies.append(pltpu.async_copy(wk_hbm.at[row0 + b], wk_v.at[b], sem))
        for cp in copies:
            cp.wait()
        for b in range(_GRP):
            def sum_body(j, a):
                for u in range(4):
                    a = a + wq_v[b, pl.ds((j * 4 + u) * _LANES, _LANES)]
                return a
            a16 = lax.fori_loop(0, nch // 4, sum_body,
                                jnp.zeros((_LANES,), jnp.float32))
            li = g * _GRP + b
            bq16 = bq_v[pl.ds((li // _LANES) * _LANES, _LANES)]
            bqv = _lane_gather(bq16, jnp.full((_LANES,), li % _LANES,
                                              jnp.int32))
            qc = allsum(a16) + bqv

            def upd_body(j, _):
                for u in range(4):
                    sl = pl.ds((j * 4 + u) * _LANES, _LANES)
                    plsc.addupdate(acc_v.at[sl], qc * wk_v[b, sl])
                return 0
            lax.fori_loop(0, nch // 4, upd_body, 0)
        return 0

    lax.fori_loop(0, _RPW // _GRP, group_body, 0)
    pltpu.sync_copy(acc_v, out_hbm.at[wid])


def _sc_qk_partials(Wq, bq, Wk):
    D = Wq.shape[1]
    mesh = plsc.VectorSubcoreMesh(core_axis_name="c", subcore_axis_name="s",
                                  num_cores=_NC)
    return functools.partial(
        pl.kernel,
        mesh=mesh,
        out_type=jax.ShapeDtypeStruct((_NW, D), jnp.float32),
        scratch_types=[
            pltpu.VMEM((_GRP, D), jnp.float32),
            pltpu.VMEM((_GRP, D), jnp.float32),
            pltpu.VMEM((_RPW,), jnp.float32),
            pltpu.VMEM((D,), jnp.float32),
            pltpu.SemaphoreType.DMA,
        ],
    )(functools.partial(_sc_qk_kernel, D=D))(Wq, bq, Wk)


def _qk_kernel(wq_ref, bq_ref, wk_ref, qk_ref):
    i = pl.program_id(0)
    qc = jnp.sum(wq_ref[...], axis=1) + bq_ref[0, :]
    part = jnp.sum(qc[:, None] * wk_ref[...], axis=0, keepdims=True)

    @pl.when(i == 0)
    def _():
        qk_ref[...] = jnp.zeros_like(qk_ref)

    qk_ref[...] += part


def _flash_kernel(h_ref, qk_ref, qksc_ref, hbar_ref,
                  acc_ref, m_ref, s_ref, qkf_ref, *, inv_scale):
    i = pl.program_id(0)
    nsteps = pl.num_programs(0)

    @pl.when(i == 0)
    def _():
        m_ref[...] = jnp.full_like(m_ref, -jnp.inf)
        s_ref[...] = jnp.zeros_like(s_ref)
        acc_ref[...] = jnp.zeros_like(acc_ref)
        qkf_ref[...] = qk_ref[...] + jnp.sum(qksc_ref[...], axis=0,
                                             keepdims=True)

    h = h_ref[...]
    qk = qkf_ref[0, :]
    t = jax.lax.dot_general(
        h, qk, (((2,), (0,)), ((), ())),
        preferred_element_type=jnp.float32, precision=_HI)
    t = t * inv_scale

    m_prev = m_ref[...]
    m_new = jnp.maximum(m_prev, jnp.max(t, axis=1, keepdims=True))
    alpha = jnp.exp(m_prev - m_new)
    p = jnp.exp(t - m_new)
    s_ref[...] = s_ref[...] * alpha + jnp.sum(p, axis=1, keepdims=True)
    pv = jax.lax.dot_general(
        p, h, (((1,), (1,)), ((0,), (0,))),
        preferred_element_type=jnp.float32)
    acc_ref[...] = acc_ref[...] * alpha + pv
    m_ref[...] = m_new

    @pl.when(i == nsteps - 1)
    def _():
        hbar_ref[...] = acc_ref[...] / s_ref[...]


def _tail_kernel(hbar_ref, wv_ref, bv_ref, we_ref, be_ref,
                 expert_ref, pmax_ref, logits_ref):
    i = pl.program_id(0)
    nsteps = pl.num_programs(0)
    hb = hbar_ref[...]
    hb_hi = hb.astype(jnp.bfloat16)
    hb_lo = (hb - hb_hi.astype(jnp.float32)).astype(jnp.bfloat16)
    hb2 = jnp.concatenate([hb_hi, hb_lo], axis=0)
    wv_hi = wv_ref[...].astype(jnp.bfloat16)
    bdim = hb.shape[0]
    rr = jax.lax.dot_general(
        hb2, wv_hi, (((1,), (1,)), ((), ())),
        preferred_element_type=jnp.float32)
    r = rr[:bdim, :] + rr[bdim:, :] + bv_ref[...]
    part = jax.lax.dot_general(
        r, we_ref[...], (((1,), (1,)), ((), ())),
        preferred_element_type=jnp.float32, precision=_HI)

    @pl.when(i == 0)
    def _():
        logits_ref[...] = jnp.zeros_like(logits_ref)

    logits_ref[...] += part

    @pl.when(i == nsteps - 1)
    def _():
        logits = logits_ref[...] + be_ref[...]
        logits_ref[...] = logits
        row_max = jnp.max(logits, axis=1, keepdims=True)
        ex = jnp.exp(logits - row_max)
        denom = jnp.sum(ex, axis=1, keepdims=True)
        pmax_ref[...] = jnp.max(ex, axis=1, keepdims=True) / denom
        bdim, edim = logits.shape
        idx = jax.lax.broadcasted_iota(jnp.int32, (bdim, edim), 1)
        am = jnp.min(jnp.where(logits == row_max, idx, edim),
                     axis=1, keepdims=True)
        expert_ref[...] = (idx == am).astype(jnp.int32)


def kernel(h_dense, Wq, bq, Wk, bk, Wv, bv, We, be):
    del bk
    B, N, D = h_dense.shape
    E = We.shape[0]
    f32 = jnp.float32

    qk_sc = _sc_qk_partials(Wq, bq, Wk)

    C1 = 512
    off1 = _SC_ROWS // C1
    qk = pl.pallas_call(
        _qk_kernel,
        grid=((D - _SC_ROWS) // C1,),
        in_specs=[
            pl.BlockSpec((C1, D), lambda i: (i + off1, 0)),
            pl.BlockSpec((1, C1), lambda i: (0, i + off1)),
            pl.BlockSpec((C1, D), lambda i: (i + off1, 0)),
        ],
        out_specs=pl.BlockSpec((1, D), lambda i: (0, 0)),
        out_shape=jax.ShapeDtypeStruct((1, D), f32),
    )(Wq, bq.reshape(1, D), Wk)

    C2 = 256
    hbar = pl.pallas_call(
        functools.partial(_flash_kernel, inv_scale=1.0 / (float(D) ** 0.5)),
        grid=(N // C2,),
        in_specs=[
            pl.BlockSpec((B, C2, D), lambda i: (0, i, 0)),
            pl.BlockSpec((1, D), lambda i: (0, 0)),
            pl.BlockSpec((_NW, D), lambda i: (0, 0)),
        ],
        out_specs=pl.BlockSpec((B, D), lambda i: (0, 0)),
        out_shape=jax.ShapeDtypeStruct((B, D), f32),
        scratch_shapes=[
            pltpu.VMEM((B, D), f32),
            pltpu.VMEM((B, 1), f32),
            pltpu.VMEM((B, 1), f32),
            pltpu.VMEM((1, D), f32),
        ],
    )(h_dense, qk, qk_sc)

    C3 = 512
    expert, pmax, logits = pl.pallas_call(
        _tail_kernel,
        grid=(D // C3,),
        in_specs=[
            pl.BlockSpec((B, D), lambda i: (0, 0)),
            pl.BlockSpec((C3, D), lambda i: (i, 0)),
            pl.BlockSpec((1, C3), lambda i: (0, i)),
            pl.BlockSpec((E, C3), lambda i: (0, i)),
            pl.BlockSpec((1, E), lambda i: (0, 0)),
        ],
        out_specs=[
            pl.BlockSpec((B, E), lambda i: (0, 0)),
            pl.BlockSpec((B, 1), lambda i: (0, 0)),
            pl.BlockSpec((B, E), lambda i: (0, 0)),
        ],
        out_shape=[
            jax.ShapeDtypeStruct((B, E), jnp.int32),
            jax.ShapeDtypeStruct((B, 1), f32),
            jax.ShapeDtypeStruct((B, E), f32),
        ],
    )(hbar, Wv, bv.reshape(1, D), We, be.reshape(1, E))

    return (expert, pmax, logits)


# speedup vs baseline: 1.4593x; 1.0029x over previous
import functools

import jax
from jax import lax
import jax.numpy as jnp
from jax.experimental import pallas as pl
from jax.experimental.pallas import tpu as pltpu
from jax.experimental.pallas import tpu_sc as plsc

_HI = jax.lax.Precision.HIGHEST

_NC = 2
_NS = 16
_NW = _NC * _NS
_LANES = 16
_SC_ROWS = 256
_RPW = _SC_ROWS // _NW
_GRP = 4
_BQL = max(_RPW, _LANES)


def _lane_gather(x, idx):
    return lax.gather(
        x, idx[:, None],
        lax.GatherDimensionNumbers(offset_dims=(), collapsed_slice_dims=(0,),
                                   start_index_map=(0,)),
        (1,), mode=lax.GatherScatterMode.PROMISE_IN_BOUNDS)


def _sc_qk_kernel(wq_hbm, bq_hbm, wk_hbm, out_hbm,
                  wq_v, wk_v, bq_v, acc_v, sem, *, D):
    wid = lax.axis_index("s") * _NC + lax.axis_index("c")
    base = wid * _RPW
    nch = D // _LANES

    pltpu.sync_copy(bq_hbm.at[pl.ds(base, _BQL)], bq_v)

    def zero_body(j, _):
        for u in range(4):
            acc_v[pl.ds((j * 4 + u) * _LANES, _LANES)] = jnp.zeros(
                (_LANES,), jnp.float32)
        return 0
    lax.fori_loop(0, nch // 4, zero_body, 0)

    lanes_iota = lax.iota(jnp.int32, _LANES)

    def allsum(x):
        for k in (1, 2, 4, 8):
            x = x + _lane_gather(x, lanes_iota ^ k)
        return x

    def group_body(g, _):
        row0 = base + g * _GRP
        copies = []
        for b in range(_GRP):
            copies.append(pltpu.async_copy(wq_hbm.at[row0 + b], wq_v.at[b], sem))
            copies.append(pltpu.async_copy(wk_hbm.at[row0 + b], wk_v.at[b], sem))
        for cp in copies:
            cp.wait()
        for b in range(_GRP):
            def sum_body(j, a):
                for u in range(4):
                    a = a + wq_v[b, pl.ds((j * 4 + u) * _LANES, _LANES)]
                return a
            a16 = lax.fori_loop(0, nch // 4, sum_body,
                                jnp.zeros((_LANES,), jnp.float32))
            li = g * _GRP + b
            bq16 = bq_v[pl.ds((li // _LANES) * _LANES, _LANES)]
            bqv = _lane_gather(bq16, jnp.full((_LANES,), li % _LANES,
                                              jnp.int32))
            qc = allsum(a16) + bqv

            def upd_body(j, _):
                for u in range(4):
                    sl = pl.ds((j * 4 + u) * _LANES, _LANES)
                    plsc.addupdate(acc_v.at[sl], qc * wk_v[b, sl])
                return 0
            lax.fori_loop(0, nch // 4, upd_body, 0)
        return 0

    lax.fori_loop(0, _RPW // _GRP, group_body, 0)
    pltpu.sync_copy(acc_v, out_hbm.at[wid])


def _sc_qk_partials(Wq, bq, Wk):
    D = Wq.shape[1]
    mesh = plsc.VectorSubcoreMesh(core_axis_name="c", subcore_axis_name="s",
                                  num_cores=_NC)
    return functools.partial(
        pl.kernel,
        mesh=mesh,
        out_type=jax.ShapeDtypeStruct((_NW, D), jnp.float32),
        scratch_types=[
            pltpu.VMEM((_GRP, D), jnp.float32),
            pltpu.VMEM((_GRP, D), jnp.float32),
            pltpu.VMEM((_BQL,), jnp.float32),
            pltpu.VMEM((D,), jnp.float32),
            pltpu.SemaphoreType.DMA,
        ],
    )(functools.partial(_sc_qk_kernel, D=D))(Wq, bq, Wk)


def _qk_kernel(wq_ref, bq_ref, wk_ref, qk_ref):
    i = pl.program_id(0)
    qc = jnp.sum(wq_ref[...], axis=1) + bq_ref[0, :]
    part = jnp.sum(qc[:, None] * wk_ref[...], axis=0, keepdims=True)

    @pl.when(i == 0)
    def _():
        qk_ref[...] = jnp.zeros_like(qk_ref)

    qk_ref[...] += part


def _flash_kernel(h_ref, qk_ref, qksc_ref, hbar_ref,
                  acc_ref, m_ref, s_ref, qkf_ref, *, inv_scale):
    i = pl.program_id(0)
    nsteps = pl.num_programs(0)

    @pl.when(i == 0)
    def _():
        m_ref[...] = jnp.full_like(m_ref, -jnp.inf)
        s_ref[...] = jnp.zeros_like(s_ref)
        acc_ref[...] = jnp.zeros_like(acc_ref)
        qkf_ref[...] = qk_ref[...] + jnp.sum(qksc_ref[...], axis=0,
                                             keepdims=True)

    h = h_ref[...]
    qk = qkf_ref[0, :]
    t = jax.lax.dot_general(
        h, qk, (((2,), (0,)), ((), ())),
        preferred_element_type=jnp.float32, precision=_HI)
    t = t * inv_scale

    m_prev = m_ref[...]
    m_new = jnp.maximum(m_prev, jnp.max(t, axis=1, keepdims=True))
    alpha = jnp.exp(m_prev - m_new)
    p = jnp.exp(t - m_new)
    s_ref[...] = s_ref[...] * alpha + jnp.sum(p, axis=1, keepdims=True)
    pv = jax.lax.dot_general(
        p, h, (((1,), (1,)), ((0,), (0,))),
        preferred_element_type=jnp.float32)
    acc_ref[...] = acc_ref[...] * alpha + pv
    m_ref[...] = m_new

    @pl.when(i == nsteps - 1)
    def _():
        hbar_ref[...] = acc_ref[...] / s_ref[...]


def _tail_kernel(hbar_ref, wv_ref, bv_ref, we_ref, be_ref,
                 expert_ref, pmax_ref, logits_ref):
    i = pl.program_id(0)
    nsteps = pl.num_programs(0)
    hb = hbar_ref[...]
    hb_hi = hb.astype(jnp.bfloat16)
    hb_lo = (hb - hb_hi.astype(jnp.float32)).astype(jnp.bfloat16)
    hb2 = jnp.concatenate([hb_hi, hb_lo], axis=0)
    wv_hi = wv_ref[...].astype(jnp.bfloat16)
    bdim = hb.shape[0]
    rr = jax.lax.dot_general(
        hb2, wv_hi, (((1,), (1,)), ((), ())),
        preferred_element_type=jnp.float32)
    r = rr[:bdim, :] + rr[bdim:, :] + bv_ref[...]
    part = jax.lax.dot_general(
        r, we_ref[...], (((1,), (1,)), ((), ())),
        preferred_element_type=jnp.float32, precision=_HI)

    @pl.when(i == 0)
    def _():
        logits_ref[...] = jnp.zeros_like(logits_ref)

    logits_ref[...] += part

    @pl.when(i == nsteps - 1)
    def _():
        logits = logits_ref[...] + be_ref[...]
        logits_ref[...] = logits
        row_max = jnp.max(logits, axis=1, keepdims=True)
        ex = jnp.exp(logits - row_max)
        denom = jnp.sum(ex, axis=1, keepdims=True)
        pmax_ref[...] = jnp.max(ex, axis=1, keepdims=True) / denom
        bdim, edim = logits.shape
        idx = jax.lax.broadcasted_iota(jnp.int32, (bdim, edim), 1)
        am = jnp.min(jnp.where(logits == row_max, idx, edim),
                     axis=1, keepdims=True)
        expert_ref[...] = (idx == am).astype(jnp.int32)


def kernel(h_dense, Wq, bq, Wk, bk, Wv, bv, We, be):
    del bk
    B, N, D = h_dense.shape
    E = We.shape[0]
    f32 = jnp.float32

    qk_sc = _sc_qk_partials(Wq, bq, Wk)

    C1 = 512
    off1 = _SC_ROWS // C1
    qk = pl.pallas_call(
        _qk_kernel,
        grid=((D - _SC_ROWS) // C1,),
        in_specs=[
            pl.BlockSpec((C1, D), lambda i: (i + off1, 0)),
            pl.BlockSpec((1, C1), lambda i: (0, i + off1)),
            pl.BlockSpec((C1, D), lambda i: (i + off1, 0)),
        ],
        out_specs=pl.BlockSpec((1, D), lambda i: (0, 0)),
        out_shape=jax.ShapeDtypeStruct((1, D), f32),
    )(Wq, bq.reshape(1, D), Wk)

    C2 = 256
    hbar = pl.pallas_call(
        functools.partial(_flash_kernel, inv_scale=1.0 / (float(D) ** 0.5)),
        grid=(N // C2,),
        in_specs=[
            pl.BlockSpec((B, C2, D), lambda i: (0, i, 0)),
            pl.BlockSpec((1, D), lambda i: (0, 0)),
            pl.BlockSpec((_NW, D), lambda i: (0, 0)),
        ],
        out_specs=pl.BlockSpec((B, D), lambda i: (0, 0)),
        out_shape=jax.ShapeDtypeStruct((B, D), f32),
        scratch_shapes=[
            pltpu.VMEM((B, D), f32),
            pltpu.VMEM((B, 1), f32),
            pltpu.VMEM((B, 1), f32),
            pltpu.VMEM((1, D), f32),
        ],
    )(h_dense, qk, qk_sc)

    C3 = 512
    expert, pmax, logits = pl.pallas_call(
        _tail_kernel,
        grid=(D // C3,),
        in_specs=[
            pl.BlockSpec((B, D), lambda i: (0, 0)),
            pl.BlockSpec((C3, D), lambda i: (i, 0)),
            pl.BlockSpec((1, C3), lambda i: (0, i)),
            pl.BlockSpec((E, C3), lambda i: (0, i)),
            pl.BlockSpec((1, E), lambda i: (0, 0)),
        ],
        out_specs=[
            pl.BlockSpec((B, E), lambda i: (0, 0)),
            pl.BlockSpec((B, 1), lambda i: (0, 0)),
            pl.BlockSpec((B, E), lambda i: (0, 0)),
        ],
        out_shape=[
            jax.ShapeDtypeStruct((B, E), jnp.int32),
            jax.ShapeDtypeStruct((B, 1), f32),
            jax.ShapeDtypeStruct((B, E), f32),
        ],
    )(hbar, Wv, bv.reshape(1, D), We, be.reshape(1, E))

    return (expert, pmax, logits)
